# baseline (speedup 1.0000x reference)
import functools

import jax
import jax.numpy as jnp
from jax import lax
from jax.experimental import pallas as pl
from jax.experimental.pallas import tpu as pltpu
from jax.experimental.pallas import tpu_sc as plsc

NC = 2
NS = 16
NW = NC * NS
LANES = 16


def _bias_body(sym_t_ref, pos_t_ref, out_ref):
    colsum = jnp.sum(sym_t_ref[...], axis=1, keepdims=True)
    bias_t = pos_t_ref[...] - colsum * (1.0 / sym_t_ref.shape[1])
    out_ref[...] = jnp.transpose(bias_t)


def _repack_body(sym_t_ref, out_ref):
    out_ref[...] = jnp.transpose(sym_t_ref[...])


@functools.partial(jax.jit, static_argnames=("B", "S", "D"))
def _sc_embed(idx_t, sym_lin, bias, *, B, S, D):
    CB = B // 2
    base_p, rem = divmod(S, NW)
    max_p = base_p + (1 if rem else 0)
    nvec = D // LANES
    mesh = plsc.VectorSubcoreMesh(
        core_axis_name="c", subcore_axis_name="s", num_cores=NC, num_subcores=NS
    )

    @functools.partial(
        pl.kernel,
        out_type=jax.ShapeDtypeStruct((S, B, D), jnp.float32),
        mesh=mesh,
        scratch_types=[
            pltpu.VMEM((max_p, B), jnp.int32),
            pltpu.VMEM((max_p, D), jnp.float32),
            pltpu.VMEM((CB, D), jnp.float32),
            pltpu.VMEM((CB, D), jnp.float32),
            pltpu.SemaphoreType.DMA,
            pltpu.SemaphoreType.DMA,
        ],
        compiler_params=pltpu.CompilerParams(use_tc_tiling_on_sc=False),
    )
    def body(idx_hbm, sym_hbm, bias_hbm, out_hbm,
             idx_v, bias_v, rowsA, rowsB, gA, gB):
        wid = lax.axis_index("s") * NC + lax.axis_index("c")
        np_ = base_p + jnp.where(wid < rem, 1, 0)
        s0 = base_p * wid + jnp.minimum(wid, rem)
        start = jnp.minimum(s0, S - max_p)
        off = s0 - start

        pltpu.sync_copy(idx_hbm.at[pl.ds(start, max_p)], idx_v)
        pltpu.sync_copy(bias_hbm.at[pl.ds(start, max_p)], bias_v)

        def gather_args(p, half, buf, sem):
            return [
                (sym_hbm.at[idx_v.at[off + p, pl.ds(half * CB + j * 128, 128)]],
                 buf.at[pl.ds(j * 128, 128)], sem)
                for j in range(CB // 128)
            ]

        def fire(p, half, buf, sem):
            for a in gather_args(p, half, buf, sem):
                pltpu.async_copy(*a)

        def wait(p, half, buf, sem):
            for a in gather_args(p, half, buf, sem):
                pltpu.make_async_copy(*a).wait()

        def add_bias(p, buf):
            def r_loop(r, carry):
                for v in range(nvec):
                    sl = pl.ds(v * LANES, LANES)
                    buf[r, sl] = buf[r, sl] + bias_v[off + p, sl]
                return carry

            lax.fori_loop(0, CB, r_loop, 0)

        def pos_body(p, carry):
            s = s0 + p
            fire(p, 0, rowsA, gA)
            fire(p, 1, rowsB, gB)
            wait(p, 0, rowsA, gA)
            add_bias(p, rowsA)
            pltpu.sync_copy(rowsA, out_hbm.at[s, pl.ds(0, CB)])
            wait(p, 1, rowsB, gB)
            add_bias(p, rowsB)
            pltpu.sync_copy(rowsB, out_hbm.at[s, pl.ds(CB, CB)])
            return carry

        lax.fori_loop(0, np_, pos_body, 0)

    return body(idx_t, sym_lin, bias)


def kernel(inputs, sym_table, pos_table):
    B, S = inputs.shape
    V, D = sym_table.shape
    sym_t = sym_table.T
    pos_t = pos_table[:S].T
    idx_t = inputs.T.astype(jnp.int32)

    bias = pl.pallas_call(
        _bias_body,
        out_shape=jax.ShapeDtypeStruct((S, D), jnp.float32),
    )(sym_t, pos_t)

    BLK = 4096
    sym_lin = pl.pallas_call(
        _repack_body,
        out_shape=jax.ShapeDtypeStruct((V, D), jnp.float32),
        grid=(pl.cdiv(V, BLK),),
        in_specs=[pl.BlockSpec((D, BLK), lambda i: (0, i))],
        out_specs=pl.BlockSpec((BLK, D), lambda i: (i, 0)),
    )(sym_t)

    out_t = _sc_embed(idx_t, sym_lin, bias, B=B, S=S, D=D)
    return out_t.transpose(1, 0, 2)


# speedup vs baseline: 1.1126x; 1.1126x over previous
import functools

import jax
import jax.numpy as jnp
from jax import lax
from jax.experimental import pallas as pl
from jax.experimental.pallas import tpu as pltpu
from jax.experimental.pallas import tpu_sc as plsc

NC = 2
NS = 16
NW = NC * NS
LANES = 16


def _bias_body(sym_t_ref, pos_t_ref, out_ref):
    colsum = jnp.sum(sym_t_ref[...], axis=1, keepdims=True)
    bias_t = pos_t_ref[...] - colsum * (1.0 / sym_t_ref.shape[1])
    out_ref[...] = jnp.transpose(bias_t)


def _repack_body(sym_t_ref, out_ref):
    out_ref[...] = jnp.transpose(sym_t_ref[...])


@functools.partial(jax.jit, static_argnames=("B", "S", "D"))
def _sc_embed(idx_t, sym_lin, bias, *, B, S, D):
    CB = 256
    NQ = B // CB
    NSLOT = 3
    base_p, rem = divmod(S, NW)
    max_p = base_p + (1 if rem else 0)
    nvec = D // LANES
    mesh = plsc.VectorSubcoreMesh(
        core_axis_name="c", subcore_axis_name="s", num_cores=NC, num_subcores=NS
    )

    @functools.partial(
        pl.kernel,
        out_type=jax.ShapeDtypeStruct((S, B, D), jnp.float32),
        mesh=mesh,
        scratch_types=[
            pltpu.VMEM((max_p, B), jnp.int32),
            pltpu.VMEM((max_p, D), jnp.float32),
            pltpu.VMEM((CB, D), jnp.float32),
            pltpu.VMEM((CB, D), jnp.float32),
            pltpu.VMEM((CB, D), jnp.float32),
            pltpu.SemaphoreType.DMA,
            pltpu.SemaphoreType.DMA,
            pltpu.SemaphoreType.DMA,
            pltpu.SemaphoreType.DMA,
            pltpu.SemaphoreType.DMA,
            pltpu.SemaphoreType.DMA,
        ],
        compiler_params=pltpu.CompilerParams(use_tc_tiling_on_sc=False),
    )
    def body(idx_hbm, sym_hbm, bias_hbm, out_hbm,
             idx_v, bias_v, rows0, rows1, rows2, g0, g1, g2, w0, w1, w2):
        wid = lax.axis_index("s") * NC + lax.axis_index("c")
        np_ = base_p + jnp.where(wid < rem, 1, 0)
        s0 = base_p * wid + jnp.minimum(wid, rem)
        nch = np_ * NQ
        start = jnp.minimum(s0, S - max_p)
        off = s0 - start
        rows = (rows0, rows1, rows2)
        gsem = (g0, g1, g2)
        wsem = (w0, w1, w2)

        pltpu.sync_copy(idx_hbm.at[pl.ds(start, max_p)], idx_v)
        pltpu.sync_copy(bias_hbm.at[pl.ds(start, max_p)], bias_v)

        def gather_args(c, slot):
            p, q = c // NQ, c % NQ
            return [
                (sym_hbm.at[idx_v.at[off + p, pl.ds(q * CB + j * 128, 128)]],
                 rows[slot].at[pl.ds(j * 128, 128)], gsem[slot])
                for j in range(CB // 128)
            ]

        def write_args(c, slot):
            p, q = c // NQ, c % NQ
            return (rows[slot], out_hbm.at[s0 + p, pl.ds(q * CB, CB)],
                    wsem[slot])

        def fire_gather(c, slot):
            for a in gather_args(c, slot):
                pltpu.async_copy(*a)

        def wait_gather(c, slot):
            for a in gather_args(c, slot):
                pltpu.make_async_copy(*a).wait()

        def compute(c, slot):
            p = c // NQ
            r_ref = rows[slot]

            def r_loop(r, carry):
                for v in range(nvec):
                    sl = pl.ds(v * LANES, LANES)
                    r_ref[r, sl] = r_ref[r, sl] + bias_v[off + p, sl]
                return carry

            lax.fori_loop(0, CB, r_loop, 0)

        def chunk_iter(c, carry):
            for k in range(NSLOT):
                cc = NSLOT * c + k
                slot = (k + 1) % NSLOT

                @pl.when(cc < nch)
                def _():
                    @pl.when(cc + 1 < nch)
                    def _():
                        @pl.when(cc >= 2)
                        def _():
                            pltpu.make_async_copy(*write_args(cc - 2, slot)).wait()

                        fire_gather(cc + 1, slot)

                    wait_gather(cc, k)
                    compute(cc, k)
                    pltpu.async_copy(*write_args(cc, k))
            return carry

        fire_gather(0, 0)
        lax.fori_loop(0, (nch + NSLOT - 1) // NSLOT, chunk_iter, 0)
        for nch_s in (base_p * NQ, (base_p + 1) * NQ):

            @pl.when(nch == nch_s)
            def _():
                for c in (nch_s - 3, nch_s - 2, nch_s - 1):
                    pltpu.make_async_copy(*write_args(c, c % NSLOT)).wait()

    return body(idx_t, sym_lin, bias)


def kernel(inputs, sym_table, pos_table):
    B, S = inputs.shape
    V, D = sym_table.shape
    sym_t = sym_table.T
    pos_t = pos_table[:S].T
    idx_t = inputs.T.astype(jnp.int32)

    bias = pl.pallas_call(
        _bias_body,
        out_shape=jax.ShapeDtypeStruct((S, D), jnp.float32),
    )(sym_t, pos_t)

    BLK = 4096
    sym_lin = pl.pallas_call(
        _repack_body,
        out_shape=jax.ShapeDtypeStruct((V, D), jnp.float32),
        grid=(pl.cdiv(V, BLK),),
        in_specs=[pl.BlockSpec((D, BLK), lambda i: (0, i))],
        out_specs=pl.BlockSpec((BLK, D), lambda i: (i, 0)),
    )(sym_t)

    out_t = _sc_embed(idx_t, sym_lin, bias, B=B, S=S, D=D)
    return out_t.transpose(1, 0, 2)
